# baseline (speedup 1.0000x reference)
import functools

import jax
import jax.numpy as jnp
from jax import lax
from jax.experimental import pallas as pl
from jax.experimental.pallas import tpu as pltpu
from jax.experimental.pallas import tpu_sc as plsc

B = 16384
DEG = 32
D = 128
LANES = 16
NC, NS = 2, 16
NW = NC * NS
BPW = B // NW
IPR = 128
NPC = IPR // DEG
NCHUNK = BPW // NPC
IDXROWS = BPW * DEG // IPR
NV = D // LANES
NBUF = 2


def _tanh_of_mean(s):
    a = jnp.exp(s * (2.0 / DEG))
    return (a - 1.0) / (a + 1.0)


@functools.partial(
    pl.kernel,
    out_type=jax.ShapeDtypeStruct((B, D), jnp.float32),
    mesh=plsc.VectorSubcoreMesh(core_axis_name="c", subcore_axis_name="s"),
    compiler_params=pltpu.CompilerParams(needs_layout_passes=False),
    scratch_types=[
        pltpu.VMEM((IDXROWS, IPR), jnp.int32),
        pltpu.VMEM((IPR, D), jnp.float32),
        pltpu.VMEM((IPR, D), jnp.float32),
        pltpu.VMEM((BPW // 2, D), jnp.float32),
        pltpu.SemaphoreType.DMA,
        pltpu.SemaphoreType.DMA,
    ],
)
def _ego_encode(idx_hbm, feat_hbm, out_hbm, idx_v, rows0, rows1,
                ostage, sem0, sem1):
    wid = lax.axis_index("s") * NC + lax.axis_index("c")
    pltpu.sync_copy(idx_hbm.at[wid], idx_v)

    rows = (rows0, rows1)
    sems = (sem0, sem1)

    def start(g, buf, sem):
        pltpu.async_copy(feat_hbm.at[idx_v.at[g]], buf, sem)

    def wait(buf, sem):
        pltpu.make_async_copy(feat_hbm.at[idx_v.at[0]], buf, sem).wait()

    def reduce_chunk(g, buf):
        for n in range(NPC):
            rbase = n * DEG

            accs = [buf[rbase, pl.ds(j * LANES, LANES)] for j in range(NV)]
            for rr in range(rbase + 1, rbase + DEG):
                for j in range(NV):
                    accs[j] = accs[j] + buf[rr, pl.ds(j * LANES, LANES)]

            half = NCHUNK // 2
            orow = jnp.where(g < half, g, g - half) * NPC + n
            for j in range(NV):
                ostage[orow, pl.ds(j * LANES, LANES)] = _tanh_of_mean(accs[j])

    for p in range(NBUF - 1):
        start(p, rows[p], sems[p])

    def outer(i, carry):
        for b in range(NBUF):
            g = NBUF * i + b
            nxt = (b + NBUF - 1) % NBUF

            @pl.when(g + NBUF - 1 < NCHUNK)
            def _(g=g, nxt=nxt):
                start(g + NBUF - 1, rows[nxt], sems[nxt])

            wait(rows[b], sems[b])
            reduce_chunk(g, rows[b])

            @pl.when(g == NCHUNK // 2 - 1)
            def _(g=g):
                pltpu.sync_copy(ostage,
                                out_hbm.at[pl.ds(wid * BPW, BPW // 2)])
        return carry

    lax.fori_loop(0, NCHUNK // NBUF, outer, 0)
    pltpu.sync_copy(ostage, out_hbm.at[pl.ds(wid * BPW + BPW // 2, BPW // 2)])


def kernel(nodes, neigh_idx, features, weight):
    del nodes, weight
    idx = neigh_idx.reshape(NW, IDXROWS, IPR)
    return _ego_encode(idx, features)


# speedup vs baseline: 2.0099x; 2.0099x over previous
import functools

import jax
import jax.numpy as jnp
from jax import lax
from jax.experimental import pallas as pl
from jax.experimental.pallas import tpu as pltpu
from jax.experimental.pallas import tpu_sc as plsc

B = 16384
DEG = 32
D = 128
LANES = 16
NC, NS = 2, 16
NW = NC * NS
BPW = B // NW
IPR = 128
NPC = IPR // DEG
NCHUNK = BPW // NPC
IDXROWS = BPW * DEG // IPR
NV = D // LANES
NBUF = 4


def _tanh_of_mean(s):
    a = jnp.exp(s * (2.0 / DEG))
    return (a - 1.0) / (a + 1.0)


@functools.partial(
    pl.kernel,
    out_type=jax.ShapeDtypeStruct((B, D), jnp.float32),
    mesh=plsc.VectorSubcoreMesh(core_axis_name="c", subcore_axis_name="s"),
    compiler_params=pltpu.CompilerParams(needs_layout_passes=False),
    scratch_types=[
        pltpu.VMEM((IDXROWS, IPR), jnp.int32),
        pltpu.VMEM((IPR, D), jnp.float32),
        pltpu.VMEM((IPR, D), jnp.float32),
        pltpu.VMEM((IPR, D), jnp.float32),
        pltpu.VMEM((IPR, D), jnp.float32),
        pltpu.VMEM((BPW // 2, D), jnp.float32),
        pltpu.SemaphoreType.DMA,
        pltpu.SemaphoreType.DMA,
        pltpu.SemaphoreType.DMA,
        pltpu.SemaphoreType.DMA,
    ],
)
def _ego_encode(idx_hbm, feat_hbm, out_hbm, idx_v, rows0, rows1, rows2, rows3,
                ostage, sem0, sem1, sem2, sem3):
    wid = lax.axis_index("s") * NC + lax.axis_index("c")
    pltpu.sync_copy(idx_hbm.at[wid], idx_v)

    rows = (rows0, rows1, rows2, rows3)
    sems = (sem0, sem1, sem2, sem3)

    def start(g, buf, sem):
        pltpu.async_copy(feat_hbm.at[idx_v.at[g]], buf, sem)

    def wait(buf, sem):
        pltpu.make_async_copy(feat_hbm.at[idx_v.at[0]], buf, sem).wait()

    def reduce_chunk(g, buf):
        for n in range(NPC):
            rbase = n * DEG

            def body(r, accs):
                new = list(accs)
                for j in range(NV):
                    new[j] = new[j] + buf[r, pl.ds(j * LANES, LANES)]
                return tuple(new)

            accs = lax.fori_loop(
                rbase, rbase + DEG, body,
                tuple(jnp.zeros((LANES,), jnp.float32) for _ in range(NV)),
            )

            half = NCHUNK // 2
            orow = jnp.where(g < half, g, g - half) * NPC + n
            for j in range(NV):
                ostage[orow, pl.ds(j * LANES, LANES)] = _tanh_of_mean(accs[j])

    for p in range(NBUF - 1):
        start(p, rows[p], sems[p])

    def outer(i, carry):
        for b in range(NBUF):
            g = NBUF * i + b
            nxt = (b + NBUF - 1) % NBUF

            @pl.when(g + NBUF - 1 < NCHUNK)
            def _(g=g, nxt=nxt):
                start(g + NBUF - 1, rows[nxt], sems[nxt])

            wait(rows[b], sems[b])
            reduce_chunk(g, rows[b])

            @pl.when(g == NCHUNK // 2 - 1)
            def _(g=g):
                pltpu.sync_copy(ostage,
                                out_hbm.at[pl.ds(wid * BPW, BPW // 2)])
        return carry

    lax.fori_loop(0, NCHUNK // NBUF, outer, 0)
    pltpu.sync_copy(ostage, out_hbm.at[pl.ds(wid * BPW + BPW // 2, BPW // 2)])


def kernel(nodes, neigh_idx, features, weight):
    del nodes, weight
    idx = neigh_idx.reshape(NW, IDXROWS, IPR)
    return _ego_encode(idx, features)
